# baseline (speedup 1.0000x reference)
import functools

import jax
import jax.numpy as jnp
from jax import lax
from jax.experimental import pallas as pl
from jax.experimental.pallas import tpu as pltpu
from jax.experimental.pallas import tpu_sc as plsc

_BATCH = 16384
_DIM = 192
_NC = 2
_NS = 16
_NW = _NC * _NS
_B_PER_W = _BATCH // _NW


def _gather_kernel(table_hbm, idx_hbm, out_hbm, idx_v, sem):
    wid = lax.axis_index("s") * _NC + lax.axis_index("c")
    base = wid * _B_PER_W
    pltpu.sync_copy(idx_hbm.at[pl.ds(base, _B_PER_W)], idx_v)

    def body(g, _):
        vec = idx_v[pl.ds(g * 16, 16)]
        for j in range(16):
            r = vec[j]
            pltpu.async_copy(
                table_hbm.at[pl.ds(r, 1)],
                out_hbm.at[pl.ds(base + g * 16 + j, 1)],
                sem,
            )
        return 0

    lax.fori_loop(0, _B_PER_W // 16, body, 0)
    pltpu.make_async_copy(
        table_hbm.at[pl.ds(0, _B_PER_W)],
        out_hbm.at[pl.ds(base, _B_PER_W)],
        sem,
    ).wait()


@jax.jit
def _route_lookup(idx, embed_weight):
    run = pl.kernel(
        _gather_kernel,
        out_type=jax.ShapeDtypeStruct((_BATCH, _DIM), jnp.float32),
        mesh=plsc.VectorSubcoreMesh(core_axis_name="c", subcore_axis_name="s"),
        scratch_types=[
            pltpu.VMEM((_B_PER_W,), jnp.int32),
            pltpu.SemaphoreType.DMA,
        ],
    )
    return run(embed_weight, idx)


def kernel(idx, embed_weight):
    return _route_lookup(idx, embed_weight)


# speedup vs baseline: 1.1787x; 1.1787x over previous
import functools

import jax
import jax.numpy as jnp
from jax import lax
from jax.experimental import pallas as pl
from jax.experimental.pallas import tpu as pltpu
from jax.experimental.pallas import tpu_sc as plsc

_BATCH = 16384
_DIM = 192
_PAD = 256
_NC = 2
_NS = 16
_NW = _NC * _NS
_B_PER_W = _BATCH // _NW
_CHUNK = 128
_NCHUNK = _B_PER_W // _CHUNK


def _gather_kernel(table_hbm, idx_hbm, out_hbm,
                   idx_v, buf0, buf1, in0, in1, out0, out1):
    wid = lax.axis_index("s") * _NC + lax.axis_index("c")
    base = wid * _B_PER_W
    pltpu.sync_copy(idx_hbm.at[pl.ds(base, _B_PER_W)], idx_v)
    bufs = (buf0, buf1)
    in_sems = (in0, in1)
    out_sems = (out0, out1)

    def gather(c):
        return pltpu.async_copy(
            table_hbm.at[idx_v.at[pl.ds(c * _CHUNK, _CHUNK)]],
            bufs[c & 1], in_sems[c & 1])

    def write(c):
        return pltpu.async_copy(
            bufs[c & 1],
            out_hbm.at[pl.ds(base + c * _CHUNK, _CHUNK)],
            out_sems[c & 1])

    g0 = gather(0)
    g1 = gather(1)
    g0.wait()
    w0 = write(0)
    g1.wait()
    w1 = write(1)
    w0.wait()
    g2 = gather(2)
    w1.wait()
    g3 = gather(3)
    g2.wait()
    w2 = write(2)
    g3.wait()
    w3 = write(3)
    w2.wait()
    w3.wait()


@jax.jit
def _route_lookup(idx, embed_weight):
    table256 = jnp.pad(embed_weight, ((0, 0), (0, _PAD - _DIM)))
    run = pl.kernel(
        _gather_kernel,
        out_type=jax.ShapeDtypeStruct((_BATCH, _PAD), jnp.float32),
        mesh=plsc.VectorSubcoreMesh(core_axis_name="c", subcore_axis_name="s"),
        scratch_types=[
            pltpu.VMEM((_B_PER_W,), jnp.int32),
            pltpu.VMEM((_CHUNK, _PAD), jnp.float32),
            pltpu.VMEM((_CHUNK, _PAD), jnp.float32),
            pltpu.SemaphoreType.DMA,
            pltpu.SemaphoreType.DMA,
            pltpu.SemaphoreType.DMA,
            pltpu.SemaphoreType.DMA,
        ],
        compiler_params=pltpu.CompilerParams(use_tc_tiling_on_sc=True),
    )
    return run(table256, idx)[:, :_DIM]


def kernel(idx, embed_weight):
    return _route_lookup(idx, embed_weight)


# speedup vs baseline: 3.1211x; 2.6480x over previous
import functools

import jax
import jax.numpy as jnp
from jax import lax
from jax.experimental import pallas as pl
from jax.experimental.pallas import tpu as pltpu
from jax.experimental.pallas import tpu_sc as plsc

_BATCH = 16384
_DIM = 192
_PAD = 256
_NC = 2
_NS = 16
_NW = _NC * _NS
_B_PER_W = _BATCH // _NW
_CHUNK = 128
_NCHUNK = _B_PER_W // _CHUNK


def _gather_kernel(table_hbm, idx_hbm, out_hbm,
                   idx_v, buf0, buf1, in0, in1, out0, out1):
    wid = lax.axis_index("s") * _NC + lax.axis_index("c")
    base = wid * _B_PER_W
    pltpu.sync_copy(idx_hbm.at[pl.ds(base, _B_PER_W)], idx_v)
    bufs = (buf0, buf1)
    in_sems = (in0, in1)
    out_sems = (out0, out1)

    def gather(c):
        return pltpu.async_copy(
            table_hbm.at[idx_v.at[pl.ds(c * _CHUNK, _CHUNK)]],
            bufs[c & 1], in_sems[c & 1])

    def write(c):
        return pltpu.async_copy(
            bufs[c & 1],
            out_hbm.at[pl.ds(base + c * _CHUNK, _CHUNK)],
            out_sems[c & 1])

    g0 = gather(0)
    g1 = gather(1)
    g0.wait()
    w0 = write(0)
    g1.wait()
    w1 = write(1)
    w0.wait()
    g2 = gather(2)
    w1.wait()
    g3 = gather(3)
    g2.wait()
    w2 = write(2)
    g3.wait()
    w3 = write(3)
    w2.wait()
    w3.wait()


_PAD_ROWS = 2000


def _pad_kernel(in_ref, out_ref):
    out_ref[:, : _DIM] = in_ref[...]
    out_ref[:, _DIM:] = jnp.zeros((_PAD_ROWS, _PAD - _DIM), jnp.float32)


def _pad_table(embed_weight):
    n = embed_weight.shape[0]
    return pl.pallas_call(
        _pad_kernel,
        grid=(n // _PAD_ROWS,),
        in_specs=[pl.BlockSpec((_PAD_ROWS, _DIM), lambda i: (i, 0))],
        out_specs=pl.BlockSpec((_PAD_ROWS, _PAD), lambda i: (i, 0)),
        out_shape=jax.ShapeDtypeStruct((n, _PAD), jnp.float32),
    )(embed_weight)


@jax.jit
def _route_lookup(idx, embed_weight):
    table256 = _pad_table(embed_weight)
    run = pl.kernel(
        _gather_kernel,
        out_type=jax.ShapeDtypeStruct((_BATCH, _PAD), jnp.float32),
        mesh=plsc.VectorSubcoreMesh(core_axis_name="c", subcore_axis_name="s"),
        scratch_types=[
            pltpu.VMEM((_B_PER_W,), jnp.int32),
            pltpu.VMEM((_CHUNK, _PAD), jnp.float32),
            pltpu.VMEM((_CHUNK, _PAD), jnp.float32),
            pltpu.SemaphoreType.DMA,
            pltpu.SemaphoreType.DMA,
            pltpu.SemaphoreType.DMA,
            pltpu.SemaphoreType.DMA,
        ],
        compiler_params=pltpu.CompilerParams(use_tc_tiling_on_sc=True),
    )
    return run(table256, idx)[:, :_DIM]


def kernel(idx, embed_weight):
    return _route_lookup(idx, embed_weight)


# speedup vs baseline: 3.1571x; 1.0115x over previous
import functools

import jax
import jax.numpy as jnp
from jax import lax
from jax.experimental import pallas as pl
from jax.experimental.pallas import tpu as pltpu
from jax.experimental.pallas import tpu_sc as plsc

_BATCH = 16384
_DIM = 192
_NC = 2
_NS = 16
_NW = _NC * _NS
_B_PER_W = _BATCH // _NW
_CHUNK = 128
_NCHUNK = _B_PER_W // _CHUNK
_PAD_ROWS = 2000


def _make_gather(width):
    def _gather_kernel(table_hbm, idx_hbm, out_hbm,
                       idx_v, buf0, buf1, in0, in1, out0, out1):
        wid = lax.axis_index("s") * _NC + lax.axis_index("c")
        base = wid * _B_PER_W
        pltpu.sync_copy(idx_hbm.at[pl.ds(base, _B_PER_W)], idx_v)
        bufs = (buf0, buf1)
        in_sems = (in0, in1)
        out_sems = (out0, out1)

        def gather(c):
            iv = idx_v.at[pl.ds(c * _CHUNK, _CHUNK)]
            if width == 128:
                src = table_hbm.at[iv, pl.ds(0, 128)]
            else:
                src = table_hbm.at[iv]
            return pltpu.async_copy(src, bufs[c & 1], in_sems[c & 1])

        def write(c):
            return pltpu.async_copy(
                bufs[c & 1],
                out_hbm.at[pl.ds(base + c * _CHUNK, _CHUNK)],
                out_sems[c & 1])

        g0 = gather(0)
        g1 = gather(1)
        g0.wait()
        w0 = write(0)
        g1.wait()
        w1 = write(1)
        w0.wait()
        g2 = gather(2)
        w1.wait()
        g3 = gather(3)
        g2.wait()
        w2 = write(2)
        g3.wait()
        w3 = write(3)
        w2.wait()
        w3.wait()

    return pl.kernel(
        _gather_kernel,
        out_type=jax.ShapeDtypeStruct((_BATCH, 128), jnp.float32),
        mesh=plsc.VectorSubcoreMesh(core_axis_name="c", subcore_axis_name="s"),
        scratch_types=[
            pltpu.VMEM((_B_PER_W,), jnp.int32),
            pltpu.VMEM((_CHUNK, 128), jnp.float32),
            pltpu.VMEM((_CHUNK, 128), jnp.float32),
            pltpu.SemaphoreType.DMA,
            pltpu.SemaphoreType.DMA,
            pltpu.SemaphoreType.DMA,
            pltpu.SemaphoreType.DMA,
        ],
        compiler_params=pltpu.CompilerParams(use_tc_tiling_on_sc=True),
    )


def _tail_kernel(in_ref, out_ref):
    out_ref[:, :64] = in_ref[:, 128:]
    out_ref[:, 64:] = jnp.zeros((_PAD_ROWS, 64), jnp.float32)


def _pack_tail(embed_weight):
    n = embed_weight.shape[0]
    return pl.pallas_call(
        _tail_kernel,
        grid=(n // _PAD_ROWS,),
        in_specs=[pl.BlockSpec((_PAD_ROWS, _DIM), lambda i: (i, 0))],
        out_specs=pl.BlockSpec((_PAD_ROWS, 128), lambda i: (i, 0)),
        out_shape=jax.ShapeDtypeStruct((n, 128), jnp.float32),
    )(embed_weight)


@jax.jit
def _route_lookup(idx, embed_weight):
    tail128 = _pack_tail(embed_weight)
    out_a = _make_gather(128)(embed_weight, idx)
    out_b = _make_gather(0)(tail128, idx)
    return jnp.concatenate([out_a, out_b[:, :_DIM - 128]], axis=1)


def kernel(idx, embed_weight):
    return _route_lookup(idx, embed_weight)


# speedup vs baseline: 4.1391x; 1.3111x over previous
import functools

import jax
import jax.numpy as jnp
from jax import lax
from jax.experimental import pallas as pl
from jax.experimental.pallas import tpu as pltpu
from jax.experimental.pallas import tpu_sc as plsc

_BATCH = 16384
_DIM = 192
_PAD = 256
_NC = 2
_NS = 16
_NW = _NC * _NS
_B_PER_W = _BATCH // _NW
_CHUNK = 128
_NCHUNK = _B_PER_W // _CHUNK
_TR_ROWS = 1024


def _gather_kernel(table_hbm, idx_hbm, out_hbm,
                   idx_v, buf0, buf1, in0, in1, out0, out1):
    wid = lax.axis_index("s") * _NC + lax.axis_index("c")
    base = wid * _B_PER_W
    pltpu.sync_copy(idx_hbm.at[pl.ds(base, _B_PER_W)], idx_v)
    bufs = (buf0, buf1)
    in_sems = (in0, in1)
    out_sems = (out0, out1)

    def gather(c):
        return pltpu.async_copy(
            table_hbm.at[idx_v.at[pl.ds(c * _CHUNK, _CHUNK)]],
            bufs[c & 1], in_sems[c & 1])

    def write(c):
        return pltpu.async_copy(
            bufs[c & 1],
            out_hbm.at[pl.ds(base + c * _CHUNK, _CHUNK)],
            out_sems[c & 1])

    g0 = gather(0)
    g1 = gather(1)
    g0.wait()
    w0 = write(0)
    g1.wait()
    w1 = write(1)
    w0.wait()
    g2 = gather(2)
    w1.wait()
    g3 = gather(3)
    g2.wait()
    w2 = write(2)
    g3.wait()
    w3 = write(3)
    w2.wait()
    w3.wait()


def _transpad_kernel(in_ref, out_ref):
    out_ref[:, : _DIM] = jnp.transpose(in_ref[...])
    out_ref[:, _DIM:] = jnp.zeros((_TR_ROWS, _PAD - _DIM), jnp.float32)


def _transpad_table(table_t):
    n = table_t.shape[1]
    return pl.pallas_call(
        _transpad_kernel,
        grid=((n + _TR_ROWS - 1) // _TR_ROWS,),
        in_specs=[pl.BlockSpec((_DIM, _TR_ROWS), lambda i: (0, i))],
        out_specs=pl.BlockSpec((_TR_ROWS, _PAD), lambda i: (i, 0)),
        out_shape=jax.ShapeDtypeStruct((n, _PAD), jnp.float32),
    )(table_t)


@jax.jit
def _route_lookup(idx, embed_weight):
    table256 = _transpad_table(jnp.transpose(embed_weight))
    run = pl.kernel(
        _gather_kernel,
        out_type=jax.ShapeDtypeStruct((_BATCH, _PAD), jnp.float32),
        mesh=plsc.VectorSubcoreMesh(core_axis_name="c", subcore_axis_name="s"),
        scratch_types=[
            pltpu.VMEM((_B_PER_W,), jnp.int32),
            pltpu.VMEM((_CHUNK, _PAD), jnp.float32),
            pltpu.VMEM((_CHUNK, _PAD), jnp.float32),
            pltpu.SemaphoreType.DMA,
            pltpu.SemaphoreType.DMA,
            pltpu.SemaphoreType.DMA,
            pltpu.SemaphoreType.DMA,
        ],
        compiler_params=pltpu.CompilerParams(use_tc_tiling_on_sc=True),
    )
    return run(table256, idx)[:, :_DIM]


def kernel(idx, embed_weight):
    return _route_lookup(idx, embed_weight)
